# baseline (device time: 41116 ns/iter reference)
import jax
import jax.numpy as jnp
from jax import lax
from jax.experimental import pallas as pl
from jax.experimental.pallas import tpu as pltpu

N_DEV = 16
M = 512
CH = M // N_DEV
NH = 2
HC = M // NH


def kernel(dy, W):
    def body(dy_ref, w_ref, out_ref, acc_ref, rs_recv,
             rs_send_sems, rs_recv_sems, ag_send_sems, ag_recv_sems):
        my = lax.axis_index("i")

        acc_ref[...] = lax.dot_general(
            dy_ref[...],
            w_ref[...],
            dimension_numbers=(((1,), (1,)), ((), ())),
            preferred_element_type=jnp.float32,
        )

        def rs_send(h, p):
            return pltpu.make_async_remote_copy(
                src_ref=acc_ref.at[pl.ds(p * CH, CH), pl.ds(h * HC, HC)],
                dst_ref=rs_recv.at[my, :, pl.ds(h * HC, HC)],
                send_sem=rs_send_sems.at[h, p],
                recv_sem=rs_recv_sems.at[h, my],
                device_id=(p,),
                device_id_type=pl.DeviceIdType.MESH,
            )

        def rs_recv_wait(h, p):
            recv = pltpu.make_async_remote_copy(
                src_ref=rs_recv.at[p, :, pl.ds(h * HC, HC)],
                dst_ref=rs_recv.at[p, :, pl.ds(h * HC, HC)],
                send_sem=rs_recv_sems.at[h, p],
                recv_sem=rs_recv_sems.at[h, p],
                device_id=(my,),
                device_id_type=pl.DeviceIdType.MESH,
            )
            recv.wait_recv()

        def ag_send(h, p):
            return pltpu.make_async_remote_copy(
                src_ref=out_ref.at[pl.ds(my * CH, CH), pl.ds(h * HC, HC)],
                dst_ref=out_ref.at[pl.ds(my * CH, CH), pl.ds(h * HC, HC)],
                send_sem=ag_send_sems.at[h, p],
                recv_sem=ag_recv_sems.at[h, my],
                device_id=(p,),
                device_id_type=pl.DeviceIdType.MESH,
            )

        def ag_recv_wait(h, p):
            recv = pltpu.make_async_remote_copy(
                src_ref=rs_recv.at[0, :, pl.ds(h * HC, HC)],
                dst_ref=out_ref.at[pl.ds(p * CH, CH), pl.ds(h * HC, HC)],
                send_sem=ag_recv_sems.at[h, p],
                recv_sem=ag_recv_sems.at[h, p],
                device_id=(my,),
                device_id_type=pl.DeviceIdType.MESH,
            )
            recv.wait_recv()

        started = []

        for h in range(NH):
            for d in range(1, N_DEV):
                p = lax.rem(my + d, N_DEV)
                rdma = rs_send(h, p)
                rdma.start()
                started.append(rdma)

        rs_recv[my] = acc_ref[pl.ds(my * CH, CH), :]

        for h in range(NH):
            for d in range(1, N_DEV):
                rs_recv_wait(h, lax.rem(my + d, N_DEV))
            out_ref[pl.ds(my * CH, CH), pl.ds(h * HC, HC)] = jnp.sum(
                rs_recv[:, :, pl.ds(h * HC, HC)], axis=0
            )
            for d in range(1, N_DEV):
                p = lax.rem(my + d, N_DEV)
                rdma = ag_send(h, p)
                rdma.start()
                started.append(rdma)

        for h in range(NH):
            for d in range(1, N_DEV):
                ag_recv_wait(h, lax.rem(my + d, N_DEV))

        for rdma in started:
            rdma.wait_send()

    out_shape = jax.ShapeDtypeStruct((M, M), jnp.float32)
    return pl.pallas_call(
        body,
        out_shape=out_shape,
        in_specs=[
            pl.BlockSpec(memory_space=pltpu.VMEM),
            pl.BlockSpec(memory_space=pltpu.VMEM),
        ],
        out_specs=pl.BlockSpec(memory_space=pltpu.VMEM),
        scratch_shapes=[
            pltpu.VMEM((M, M), jnp.float32),
            pltpu.VMEM((N_DEV, CH, M), jnp.float32),
            pltpu.SemaphoreType.DMA((NH, N_DEV)),
            pltpu.SemaphoreType.DMA((NH, N_DEV)),
            pltpu.SemaphoreType.DMA((NH, N_DEV)),
            pltpu.SemaphoreType.DMA((NH, N_DEV)),
        ],
    )(dy, W)


# device time: 32689 ns/iter; 1.2578x vs baseline; 1.2578x over previous
import jax
import jax.numpy as jnp
from jax import lax
from jax.experimental import pallas as pl
from jax.experimental.pallas import tpu as pltpu

N_DEV = 16
M = 512
CH = M // N_DEV
NB = 4
BR = M // NB


def kernel(dy, W):
    def body(dy_ref, w_ref, out_ref, acc_ref, rs_recv,
             rs_send_sems, rs_recv_sems, ag_send_sems, ag_recv_sems):
        my = lax.axis_index("i")

        barrier_sem = pltpu.get_barrier_semaphore()
        for nbr in (lax.rem(my + 1, N_DEV), lax.rem(my + N_DEV - 1, N_DEV)):
            pl.semaphore_signal(
                barrier_sem, inc=1,
                device_id=(nbr,), device_id_type=pl.DeviceIdType.MESH,
            )
        pl.semaphore_wait(barrier_sem, 2)

        rs_rdmas = []
        for b in range(NB):
            rows = slice(b * BR, (b + 1) * BR)
            acc_ref[rows, :] = lax.dot_general(
                dy_ref[rows, :],
                w_ref[...],
                dimension_numbers=(((1,), (1,)), ((), ())),
                preferred_element_type=jnp.float32,
            )
            for c in range(b * NB, (b + 1) * NB):
                rdma = pltpu.make_async_remote_copy(
                    src_ref=acc_ref.at[pl.ds(c * CH, CH), :],
                    dst_ref=rs_recv.at[my],
                    send_sem=rs_send_sems.at[c],
                    recv_sem=rs_recv_sems.at[my],
                    device_id=(c,),
                    device_id_type=pl.DeviceIdType.MESH,
                )
                rs_rdmas.append((c, rdma))

                @pl.when(c != my)
                def _():
                    rdma.start()

        rs_recv[my] = acc_ref[pl.ds(my * CH, CH), :]
        for d in range(1, N_DEV):
            p = lax.rem(my + d, N_DEV)
            recv = pltpu.make_async_remote_copy(
                src_ref=rs_recv.at[p],
                dst_ref=rs_recv.at[p],
                send_sem=rs_recv_sems.at[p],
                recv_sem=rs_recv_sems.at[p],
                device_id=(my,),
                device_id_type=pl.DeviceIdType.MESH,
            )
            recv.wait_recv()
        out_ref[pl.ds(my * CH, CH), :] = jnp.sum(rs_recv[...], axis=0)

        ag_rdmas = []
        for d in range(1, N_DEV):
            p = lax.rem(my + d, N_DEV)
            rdma = pltpu.make_async_remote_copy(
                src_ref=out_ref.at[pl.ds(my * CH, CH), :],
                dst_ref=out_ref.at[pl.ds(my * CH, CH), :],
                send_sem=ag_send_sems.at[p],
                recv_sem=ag_recv_sems.at[my],
                device_id=(p,),
                device_id_type=pl.DeviceIdType.MESH,
            )
            rdma.start()
            ag_rdmas.append(rdma)

        for d in range(1, N_DEV):
            p = lax.rem(my + d, N_DEV)
            recv = pltpu.make_async_remote_copy(
                src_ref=rs_recv.at[0],
                dst_ref=out_ref.at[pl.ds(p * CH, CH), :],
                send_sem=ag_recv_sems.at[p],
                recv_sem=ag_recv_sems.at[p],
                device_id=(my,),
                device_id_type=pl.DeviceIdType.MESH,
            )
            recv.wait_recv()

        for c, rdma in rs_rdmas:
            @pl.when(c != my)
            def _():
                rdma.wait_send()
        for rdma in ag_rdmas:
            rdma.wait_send()

    out_shape = jax.ShapeDtypeStruct((M, M), jnp.float32)
    return pl.pallas_call(
        body,
        out_shape=out_shape,
        in_specs=[
            pl.BlockSpec(memory_space=pltpu.VMEM),
            pl.BlockSpec(memory_space=pltpu.VMEM),
        ],
        out_specs=pl.BlockSpec(memory_space=pltpu.VMEM),
        scratch_shapes=[
            pltpu.VMEM((M, M), jnp.float32),
            pltpu.VMEM((N_DEV, CH, M), jnp.float32),
            pltpu.SemaphoreType.DMA((N_DEV,)),
            pltpu.SemaphoreType.DMA((N_DEV,)),
            pltpu.SemaphoreType.DMA((N_DEV,)),
            pltpu.SemaphoreType.DMA((N_DEV,)),
        ],
        compiler_params=pltpu.CompilerParams(collective_id=0),
    )(dy, W)
